# Initial kernel scaffold; baseline (speedup 1.0000x reference)
import jax
import jax.numpy as jnp
from jax.experimental import pallas as pl


def kernel(q, k, v, self_indices, neighbor_indices, Wq, bq, Wk, bk, Wv, bv, Wo, bo):
    raise NotImplementedError("write your pallas kernel here")


# speedup vs baseline: 776.2010x; 776.2010x over previous
import functools
import math

import jax
import jax.numpy as jnp
from jax import lax
from jax.experimental import pallas as pl
from jax.experimental.pallas import tpu as pltpu
from jax.experimental.pallas import tpu_sc as plsc

_NC = 2
_NS = 16
_NW = _NC * _NS
_LANES = 16


def _edge_mask_sc(self_indices, n_nodes):
    e = self_indices.shape[0]
    e_per_w = e // _NW
    mesh = plsc.VectorSubcoreMesh(core_axis_name="c", subcore_axis_name="s")

    @functools.partial(
        pl.kernel,
        mesh=mesh,
        out_type=jax.ShapeDtypeStruct((_NW, n_nodes), jnp.float32),
        scratch_types=[
            pltpu.VMEM((e_per_w,), jnp.int32),
            pltpu.VMEM((n_nodes,), jnp.float32),
        ],
        compiler_params=pltpu.CompilerParams(needs_layout_passes=False),
    )
    def mask_kernel(idx_hbm, out_hbm, idx_v, acc_v):
        wid = lax.axis_index("s") * _NC + lax.axis_index("c")
        base = wid * e_per_w
        pltpu.sync_copy(idx_hbm.at[pl.ds(base, e_per_w)], idx_v)
        zeros = jnp.zeros((_LANES,), jnp.float32)
        ones = jnp.ones((_LANES,), jnp.float32)

        def zero_body(i, carry):
            acc_v[pl.ds(i * _LANES, _LANES)] = zeros
            return carry

        lax.fori_loop(0, n_nodes // _LANES, zero_body, 0)

        def scatter_body(i, carry):
            iv = idx_v[pl.ds(i * _LANES, _LANES)]
            plsc.addupdate_scatter(acc_v, [iv], ones)
            return carry

        lax.fori_loop(0, e_per_w // _LANES, scatter_body, 0)
        pltpu.sync_copy(acc_v, out_hbm.at[wid])

    return mask_kernel(self_indices)


def _proj_tc_body(v_ref, wv_ref, bv_ref, wo_ref, bo_ref, m_ref, o_ref):
    counts = lax.dot_general(
        m_ref[...],
        jnp.ones((_NW, v_ref.shape[1]), jnp.float32),
        dimension_numbers=(((0,), (0,)), ((), ())),
        preferred_element_type=jnp.float32,
    )
    gate = jnp.where(counts > 0.5, 1.0, 0.0)
    vproj = (
        lax.dot_general(
            v_ref[...],
            wv_ref[...],
            dimension_numbers=(((1,), (1,)), ((), ())),
            preferred_element_type=jnp.float32,
        )
        + bv_ref[...]
    )
    o_ref[...] = (
        lax.dot_general(
            vproj * gate,
            wo_ref[...],
            dimension_numbers=(((1,), (1,)), ((), ())),
            preferred_element_type=jnp.float32,
        )
        + bo_ref[...]
    )


def kernel(q, k, v, self_indices, neighbor_indices, Wq, bq, Wk, bk, Wv, bv, Wo, bo):
    n, d = v.shape
    blk = 2048
    n_pad = -(-n // blk) * blk
    masks = _edge_mask_sc(self_indices, n_pad)
    v_p = jnp.pad(v, ((0, n_pad - n), (0, 0)))

    out = pl.pallas_call(
        _proj_tc_body,
        grid=(n_pad // blk,),
        in_specs=[
            pl.BlockSpec((blk, d), lambda i: (i, 0)),
            pl.BlockSpec((d, d), lambda i: (0, 0)),
            pl.BlockSpec((1, d), lambda i: (0, 0)),
            pl.BlockSpec((d, d), lambda i: (0, 0)),
            pl.BlockSpec((1, d), lambda i: (0, 0)),
            pl.BlockSpec((_NW, blk), lambda i: (0, i)),
        ],
        out_specs=pl.BlockSpec((blk, d), lambda i: (i, 0)),
        out_shape=jax.ShapeDtypeStruct((n_pad, d), jnp.float32),
    )(v_p, Wv, bv.reshape(1, d), Wo, bo.reshape(1, d), masks)
    return out[:n]
